# baseline (speedup 1.0000x reference)
import jax
import jax.numpy as jnp
from jax import lax
from jax.experimental import pallas as pl
from jax.experimental.pallas import tpu as pltpu
from jax.experimental.pallas import tpu_sc as plsc

D_MODEL = 64
SCALE = 8.0
CHUNK = 128
LANES = 16
NW = 32


def _body(idx_hbm, table_hbm, out_hbm, idx_v, gbufs, obufs, pairbufs, gsems, osems):
    nc = 2
    wid = lax.axis_index("s") * nc + lax.axis_index("c")
    n_chunks = idx_hbm.shape[1]
    base = wid * n_chunks * CHUNK

    pltpu.sync_copy(idx_hbm.at[wid], idx_v)

    def prep_pair(c, b):
        for k in range(CHUNK // LANES):
            v = idx_v[c, pl.ds(LANES * k, LANES)]
            pairbufs[b][pl.ds(LANES * k, LANES)] = jnp.right_shift(v, 1)

    def gather(b):
        return pltpu.make_async_copy(table_hbm.at[pairbufs[b]], gbufs[b], gsems[b])

    def copy_out(c, b):
        return pltpu.make_async_copy(
            obufs[b], out_hbm.at[pl.ds(base + c * CHUNK, CHUNK)], osems[b])

    def select_scale(c, b):
        for k in range(CHUNK // LANES):
            iv = idx_v[c, pl.ds(LANES * k, LANES)]
            voff = jnp.left_shift(jnp.bitwise_and(iv, 1), 6)
            rvec = lax.iota(jnp.int32, LANES) + (LANES * k)

            @plsc.parallel_loop(0, D_MODEL, unroll=4)
            def _(e):
                evec = jnp.full((LANES,), 0, jnp.int32) + e
                vals = plsc.load_gather(gbufs[b], [rvec, voff + evec])
                plsc.store_scatter(obufs[b], [rvec, evec], vals * SCALE)

    for b in range(2):
        prep_pair(b, b)
        gather(b).start()

    for b in range(2):
        gather(b).wait()
        select_scale(b, b)
        copy_out(b, b).start()
        prep_pair(b + 2, b)
        gather(b).start()

    def group(i, carry):
        for b in range(2):
            c = 2 * i + b
            gather(b).wait()
            copy_out(c - 2, b).wait()
            select_scale(c, b)
            copy_out(c, b).start()
            prep_pair(c + 2, b)
            gather(b).start()
        return carry

    lax.fori_loop(1, n_chunks // 2 - 1, group, 0)

    for b in range(2):
        c = n_chunks - 2 + b
        gather(b).wait()
        copy_out(c - 2, b).wait()
        select_scale(c, b)
        copy_out(c, b).start()
    for b in range(2):
        copy_out(n_chunks - 2 + b, b).wait()


def kernel(x, embed_weight):
    n_x_rows, row_len = x.shape
    n = n_x_rows * row_len
    xi = x.astype(jnp.int32)
    idx3 = xi.reshape(NW, n // (NW * CHUNK), CHUNK)
    table2 = embed_weight.reshape(embed_weight.shape[0] // 2, 2 * D_MODEL)

    mesh = plsc.VectorSubcoreMesh(core_axis_name="c", subcore_axis_name="s")

    def body(idx_hbm, table_hbm, out_hbm, idx_v, *scratch):
        gbufs = scratch[0:2]
        obufs = scratch[2:4]
        pairbufs = scratch[4:6]
        gsems = scratch[6:8]
        osems = scratch[8:10]
        _body(idx_hbm, table_hbm, out_hbm, idx_v, gbufs, obufs, pairbufs,
              gsems, osems)

    run = pl.kernel(
        body,
        out_type=jax.ShapeDtypeStruct((n, D_MODEL), jnp.float32),
        mesh=mesh,
        scratch_types=(
            [pltpu.VMEM((n // (NW * CHUNK), CHUNK), jnp.int32)]
            + [pltpu.VMEM((CHUNK, 2 * D_MODEL), jnp.float32) for _ in range(2)]
            + [pltpu.VMEM((CHUNK, D_MODEL), jnp.float32) for _ in range(2)]
            + [pltpu.VMEM((CHUNK,), jnp.int32) for _ in range(2)]
            + [pltpu.SemaphoreType.DMA for _ in range(4)]
        ),
        compiler_params=pltpu.CompilerParams(needs_layout_passes=False),
    )
    out = run(idx3, table2)
    return out.reshape(n_x_rows, row_len, D_MODEL)


# speedup vs baseline: 1.6373x; 1.6373x over previous
import jax
import jax.numpy as jnp
from jax import lax
from jax.experimental import pallas as pl
from jax.experimental.pallas import tpu as pltpu
from jax.experimental.pallas import tpu_sc as plsc

D_MODEL = 64
SCALE = 8.0
CHUNK = 128
LANES = 16
NW = 32


def _body(idx_hbm, table_hbm, out_hbm, idx_v, gbufs, obufs, pairbufs, gsems, osems):
    nc = 2
    wid = lax.axis_index("s") * nc + lax.axis_index("c")
    n_chunks = idx_hbm.shape[1]
    base = wid * n_chunks * CHUNK

    pltpu.sync_copy(idx_hbm.at[wid], idx_v)

    def prep_pair(c, b):
        for k in range(CHUNK // LANES):
            v = idx_v[c, pl.ds(LANES * k, LANES)]
            pairbufs[b][pl.ds(LANES * k, LANES)] = jnp.right_shift(v, 1)

    def gather(b):
        return pltpu.make_async_copy(table_hbm.at[pairbufs[b]], gbufs[b], gsems[b])

    def copy_out(c, b):
        return pltpu.make_async_copy(
            obufs[b], out_hbm.at[pl.ds(base + c * CHUNK, CHUNK)], osems[b])

    def select_scale(c, b):
        @plsc.parallel_loop(0, CHUNK // LANES, unroll=2)
        def _(k):
            iv = idx_v[c, pl.ds(k * LANES, LANES)]
            offs = jnp.left_shift(jnp.bitwise_and(iv, 1), 6)
            for j in range(LANES):
                r = k * LANES + j
                off = offs[j]
                for l in range(D_MODEL // LANES):
                    obufs[b][r, pl.ds(l * LANES, LANES)] = (
                        gbufs[b][r, pl.ds(off + l * LANES, LANES)] * SCALE)

    for b in range(2):
        prep_pair(b, b)
        gather(b).start()

    for b in range(2):
        gather(b).wait()
        select_scale(b, b)
        copy_out(b, b).start()
        prep_pair(b + 2, b)
        gather(b).start()

    def group(i, carry):
        for b in range(2):
            c = 2 * i + b
            gather(b).wait()
            copy_out(c - 2, b).wait()
            select_scale(c, b)
            copy_out(c, b).start()
            prep_pair(c + 2, b)
            gather(b).start()
        return carry

    lax.fori_loop(1, n_chunks // 2 - 1, group, 0)

    for b in range(2):
        c = n_chunks - 2 + b
        gather(b).wait()
        copy_out(c - 2, b).wait()
        select_scale(c, b)
        copy_out(c, b).start()
    for b in range(2):
        copy_out(n_chunks - 2 + b, b).wait()


def kernel(x, embed_weight):
    n_x_rows, row_len = x.shape
    n = n_x_rows * row_len
    xi = x.astype(jnp.int32)
    idx3 = xi.reshape(NW, n // (NW * CHUNK), CHUNK)
    table2 = embed_weight.reshape(embed_weight.shape[0] // 2, 2 * D_MODEL)

    mesh = plsc.VectorSubcoreMesh(core_axis_name="c", subcore_axis_name="s")

    def body(idx_hbm, table_hbm, out_hbm, idx_v, *scratch):
        gbufs = scratch[0:2]
        obufs = scratch[2:4]
        pairbufs = scratch[4:6]
        gsems = scratch[6:8]
        osems = scratch[8:10]
        _body(idx_hbm, table_hbm, out_hbm, idx_v, gbufs, obufs, pairbufs,
              gsems, osems)

    run = pl.kernel(
        body,
        out_type=jax.ShapeDtypeStruct((n, D_MODEL), jnp.float32),
        mesh=mesh,
        scratch_types=(
            [pltpu.VMEM((n // (NW * CHUNK), CHUNK), jnp.int32)]
            + [pltpu.VMEM((CHUNK, 2 * D_MODEL), jnp.float32) for _ in range(2)]
            + [pltpu.VMEM((CHUNK, D_MODEL), jnp.float32) for _ in range(2)]
            + [pltpu.VMEM((CHUNK,), jnp.int32) for _ in range(2)]
            + [pltpu.SemaphoreType.DMA for _ in range(4)]
        ),
        compiler_params=pltpu.CompilerParams(needs_layout_passes=False),
    )
    out = run(idx3, table2)
    return out.reshape(n_x_rows, row_len, D_MODEL)
